# Initial kernel scaffold; baseline (speedup 1.0000x reference)
import jax
import jax.numpy as jnp
from jax.experimental import pallas as pl


def kernel(deepgo, x, u, edge, edge_attr, batch, params):
    raise NotImplementedError("write your pallas kernel here")


# speedup vs baseline: 1.4121x; 1.4121x over previous
import functools

import jax
import jax.numpy as jnp
from jax.experimental import pallas as pl

B, C = 50, 1000
BE = 4000


def _edge_block_kernel(gr_ref, gc_ref, eac_ref, a_ref, h_ref, z_ref, *, Hn1):
    gr = gr_ref[...]
    gc = gc_ref[...]
    eac = eac_ref[...]
    a = a_ref[...]
    h = jnp.maximum(gr[:, :5] + gc + eac, 0.0)
    z = jnp.maximum(gr[:, 5:] + jax.lax.dot(h, a), 0.0)
    h_ref[...] = h
    z_ref[...] = z


def _edge_compute(gr, gc, eac, a, Hn1):
    E = gr.shape[0]
    grid = (E // BE,)
    return pl.pallas_call(
        functools.partial(_edge_block_kernel, Hn1=Hn1),
        grid=grid,
        in_specs=[
            pl.BlockSpec((BE, 5 + Hn1), lambda i: (i, 0)),
            pl.BlockSpec((BE, 5), lambda i: (i, 0)),
            pl.BlockSpec((BE, 5), lambda i: (i, 0)),
            pl.BlockSpec((5, Hn1), lambda i: (0, 0)),
        ],
        out_specs=[
            pl.BlockSpec((BE, 5), lambda i: (i, 0)),
            pl.BlockSpec((BE, Hn1), lambda i: (i, 0)),
        ],
        out_shape=[
            jax.ShapeDtypeStruct((E, 5), jnp.float32),
            jax.ShapeDtypeStruct((E, Hn1), jnp.float32),
        ],
    )(gr, gc, eac, a)


def _mlp(h, p, name):
    h = jnp.maximum(h @ p[name + "_w1"] + p[name + "_b1"], 0.0)
    return h @ p[name + "_w2"] + p[name + "_b2"]


def _layer(prefix, x, row, col, eac, u, p, cnt_inv, pos, Fx, Fa, Hn1):
    W1 = p[prefix + "_e_w1"]; b1 = p[prefix + "_e_b1"]
    W2e = p[prefix + "_e_w2"]; b2e = p[prefix + "_e_b2"]
    V1 = p[prefix + "_n1_w1"]; b1n = p[prefix + "_n1_b1"]
    V2 = p[prefix + "_n1_w2"]; b2n = p[prefix + "_n1_b2"]
    W1_r, W1_c, W1_u = W1[:Fx], W1[Fx:2 * Fx], W1[2 * Fx + Fa:]
    V1_x, V1_a = V1[:Fx], V1[Fx:]

    U1bc = jnp.repeat(u @ W1_u, C, axis=0)
    PRe = x @ W1_r + U1bc + b1
    PRn = x @ V1_x + (b1n + b2e @ V1_a)
    PCe = x @ W1_c
    A = W2e @ V1_a

    PR = jnp.concatenate([PRe, PRn], axis=1)
    gr = PR[row]
    gc = PCe[col]
    h, z = _edge_compute(gr, gc, eac, A, Hn1)

    N = x.shape[0]
    Sz = jax.ops.segment_sum(z, col, num_segments=N)
    Sh = jax.ops.segment_sum(h, col, num_segments=N)
    agg = (Sz * cnt_inv) @ V2 + b2n * pos
    edge_u = (Sh * cnt_inv) @ W2e + b2e * pos

    xu = jnp.repeat(u, C, axis=0)
    x_new = _mlp(jnp.concatenate([x, agg, xu], axis=1), p, prefix + "_n2")
    return x_new, h, edge_u


def _global_model(prefix, u, x_new, edge_u, p):
    xbar = x_new.reshape(B, C, -1).mean(axis=1)
    ebar = edge_u.reshape(B, C, -1).mean(axis=1)
    return _mlp(jnp.concatenate([u, xbar, ebar], axis=1), p, prefix + "_g")


def kernel(deepgo, x, u, edge, edge_attr, batch, params):
    p = params
    nB = deepgo.shape[0]
    N = x.shape[0]
    row, col = edge[0], edge[1]
    node_dpg = jnp.squeeze(deepgo, 1).reshape(C * nB, 1)
    uu = jnp.squeeze(u[:, :, 1:21], 1)

    cnt = jax.ops.segment_sum(jnp.ones((row.shape[0], 1), jnp.float32), col,
                              num_segments=N)
    cnt_inv = 1.0 / jnp.maximum(cnt, 1.0)
    pos = jnp.where(cnt > 0, 1.0, 0.0)

    W1a_1 = p["ml1_e_w1"][2 * 12:2 * 12 + 1]
    eac1 = edge_attr @ W1a_1
    x1, h1, eu1 = _layer("ml1", x, row, col, eac1, uu, p, cnt_inv, pos, 12, 1, 15)
    uu = _global_model("ml1", uu, x1, eu1, p)

    W1_2 = p["ml2_e_w1"]
    W1a_2 = W1_2[2 * 20:2 * 20 + 10]
    eac2 = h1 @ (p["ml1_e_w2"] @ W1a_2) + (p["ml1_e_b2"] @ W1a_2)
    x2, h2, eu2 = _layer("ml2", x1, row, col, eac2, uu, p, cnt_inv, pos, 20, 10, 20)
    uu = _global_model("ml2", uu, x2, eu2, p)

    x2b = jnp.concatenate([x2, node_dpg, x[:, 10:]], axis=1)
    W1_5 = p["ml5_e_w1"]
    W1a_5 = W1_5[2 * 23:2 * 23 + 10]
    eac5 = h2 @ (p["ml2_e_w2"] @ W1a_5) + (p["ml2_e_b2"] @ W1a_5)
    x3, _, _ = _layer("ml5", x2b, row, col, eac5, uu, p, cnt_inv, pos, 23, 10, 23)
    return x3.reshape(nB, 1, -1)


# speedup vs baseline: 2.3002x; 1.6289x over previous
import functools

import jax
import jax.numpy as jnp
from jax import lax
from jax.experimental import pallas as pl
from jax.experimental.pallas import tpu as pltpu
from jax.experimental.pallas import tpu_sc as plsc

B, C = 50, 1000
BE = 4000

NC, NS = 2, 16
NW = NC * NS
GU = 128
G = 5
KB = GU * G
NPAD = 50048


def _sc_scatter_call(payload, col2d, zeros):
    E, Wp = payload.shape
    npad = NPAD
    units = E // KB
    jmax = (units + NW - 1) // NW
    rows_per_sub = npad // NS
    chunks = []
    off = 0
    while off < rows_per_sub:
        sz = min(KB, rows_per_sub - off)
        chunks.append((off, sz))
        off += sz
    mesh = plsc.VectorSubcoreMesh(core_axis_name="c", subcore_axis_name="s")

    @functools.partial(
        pl.kernel,
        out_type=jax.ShapeDtypeStruct((NC * npad, Wp), jnp.float32),
        mesh=mesh,
        scratch_types=[
            pltpu.VMEM((G, GU), jnp.int32),
            pltpu.VMEM((KB, Wp), jnp.float32),
            pltpu.VMEM_SHARED((npad, Wp), jnp.float32),
        ],
        compiler_params=pltpu.CompilerParams(use_tc_tiling_on_sc=False),
    )
    def k(pay_hbm, col_hbm, zero_hbm, out_hbm, colbuf, paybuf, acc):
        c = lax.axis_index("c")
        s = lax.axis_index("s")
        w = s * NC + c
        rs = s * rows_per_sub
        pltpu.sync_copy(zero_hbm, paybuf)
        for off, sz in chunks:
            pltpu.sync_copy(paybuf.at[pl.ds(0, sz)],
                            acc.at[pl.ds(rs + off, sz)])
        plsc.subcore_barrier()

        def body(j, carry):
            u = w + NW * j

            @pl.when(u < units)
            def _():
                pltpu.sync_copy(col_hbm.at[u], colbuf)
                pltpu.sync_copy(pay_hbm.at[pl.ds(u * KB, KB)], paybuf)
                for g in range(G):
                    pltpu.sync_copy(paybuf.at[pl.ds(g * GU, GU)],
                                    acc.at[colbuf.at[g]], add=True)

            return carry

        lax.fori_loop(0, jmax, body, 0)
        plsc.subcore_barrier()
        for off, sz in chunks:
            pltpu.sync_copy(acc.at[pl.ds(rs + off, sz)],
                            paybuf.at[pl.ds(0, sz)])
            pltpu.sync_copy(paybuf.at[pl.ds(0, sz)],
                            out_hbm.at[pl.ds(c * npad + rs + off, sz)])

    return k(payload, col2d, zeros)


def _edge_block_kernel(gr_ref, gc_ref, eac_ref, a_ref, pay_ref, *, Hn1, Wp,
                       ones_col):
    gr = gr_ref[...]
    gc = gc_ref[...]
    eac = eac_ref[...]
    a = a_ref[...]
    h = jnp.maximum(gr[:, :5] + gc + eac, 0.0)
    z = jnp.maximum(gr[:, 5:] + jax.lax.dot(h, a), 0.0)
    pad = Wp - 5 - Hn1
    parts = [h, z]
    if ones_col:
        parts.append(jnp.ones((h.shape[0], 1), jnp.float32))
        pad -= 1
    if pad:
        parts.append(jnp.zeros((h.shape[0], pad), jnp.float32))
    pay_ref[...] = jnp.concatenate(parts, axis=1)


def _edge_compute(gr, gc, eac, a, Hn1, Wp, ones_col):
    E = gr.shape[0]
    grid = (E // BE,)
    return pl.pallas_call(
        functools.partial(_edge_block_kernel, Hn1=Hn1, Wp=Wp,
                          ones_col=ones_col),
        grid=grid,
        in_specs=[
            pl.BlockSpec((BE, 5 + Hn1), lambda i: (i, 0)),
            pl.BlockSpec((BE, 5), lambda i: (i, 0)),
            pl.BlockSpec((BE, 5), lambda i: (i, 0)),
            pl.BlockSpec((5, Hn1), lambda i: (0, 0)),
        ],
        out_specs=pl.BlockSpec((BE, Wp), lambda i: (i, 0)),
        out_shape=jax.ShapeDtypeStruct((E, Wp), jnp.float32),
    )(gr, gc, eac, a)


def _mlp(h, p, name):
    h = jnp.maximum(h @ p[name + "_w1"] + p[name + "_b1"], 0.0)
    return h @ p[name + "_w2"] + p[name + "_b2"]


def _layer(prefix, x, row, col2d, eac, u, p, cnt_inv, pos, Fx, Fa, Hn1,
           Wp, zeros, ones_col=False):
    W1 = p[prefix + "_e_w1"]; b1 = p[prefix + "_e_b1"]
    W2e = p[prefix + "_e_w2"]; b2e = p[prefix + "_e_b2"]
    V1 = p[prefix + "_n1_w1"]; b1n = p[prefix + "_n1_b1"]
    V2 = p[prefix + "_n1_w2"]; b2n = p[prefix + "_n1_b2"]
    W1_r, W1_c, W1_u = W1[:Fx], W1[Fx:2 * Fx], W1[2 * Fx + Fa:]
    V1_x, V1_a = V1[:Fx], V1[Fx:]

    U1bc = jnp.repeat(u @ W1_u, C, axis=0)
    PRe = x @ W1_r + U1bc + b1
    PRn = x @ V1_x + (b1n + b2e @ V1_a)
    PCe = x @ W1_c
    A = W2e @ V1_a

    PR = jnp.concatenate([PRe, PRn], axis=1)
    gr = PR[row]
    gc = PCe[col2d.reshape(-1)]
    payload = _edge_compute(gr, gc, eac, A, Hn1, Wp, ones_col)
    h = payload[:, :5]

    N = x.shape[0]
    parts = _sc_scatter_call(payload, col2d, zeros)
    S = parts[:N] + parts[NPAD:NPAD + N]
    Sh, Sz = S[:, :5], S[:, 5:5 + Hn1]
    if ones_col:
        cnt = S[:, 5 + Hn1:6 + Hn1]
        cnt_inv = 1.0 / jnp.maximum(cnt, 1.0)
        pos = jnp.where(cnt > 0, 1.0, 0.0)
    agg = (Sz * cnt_inv) @ V2 + b2n * pos
    edge_u = (Sh * cnt_inv) @ W2e + b2e * pos

    xu = jnp.repeat(u, C, axis=0)
    x_new = _mlp(jnp.concatenate([x, agg, xu], axis=1), p, prefix + "_n2")
    return x_new, h, edge_u, cnt_inv, pos


def _global_model(prefix, u, x_new, edge_u, p):
    xbar = x_new.reshape(B, C, -1).mean(axis=1)
    ebar = edge_u.reshape(B, C, -1).mean(axis=1)
    return _mlp(jnp.concatenate([u, xbar, ebar], axis=1), p, prefix + "_g")


def kernel(deepgo, x, u, edge, edge_attr, batch, params):
    p = params
    nB = deepgo.shape[0]
    N = x.shape[0]
    row, col = edge[0], edge[1]
    col2d = col.reshape(-1, G, GU)
    node_dpg = jnp.squeeze(deepgo, 1).reshape(C * nB, 1)
    uu = jnp.squeeze(u[:, :, 1:21], 1)

    zeros24 = jnp.zeros((KB, 24), jnp.float32)
    zeros32 = jnp.zeros((KB, 32), jnp.float32)

    W1a_1 = p["ml1_e_w1"][2 * 12:2 * 12 + 1]
    eac1 = edge_attr @ W1a_1
    x1, h1, eu1, cnt_inv, pos = _layer("ml1", x, row, col2d, eac1, uu, p,
                                       None, None, 12, 1, 15, 24, zeros24,
                                       ones_col=True)
    uu = _global_model("ml1", uu, x1, eu1, p)

    W1a_2 = p["ml2_e_w1"][2 * 20:2 * 20 + 10]
    eac2 = h1 @ (p["ml1_e_w2"] @ W1a_2) + (p["ml1_e_b2"] @ W1a_2)
    x2, h2, eu2, _, _ = _layer("ml2", x1, row, col2d, eac2, uu, p,
                               cnt_inv, pos, 20, 10, 20, 32, zeros32)
    uu = _global_model("ml2", uu, x2, eu2, p)

    x2b = jnp.concatenate([x2, node_dpg, x[:, 10:]], axis=1)
    W1a_5 = p["ml5_e_w1"][2 * 23:2 * 23 + 10]
    eac5 = h2 @ (p["ml2_e_w2"] @ W1a_5) + (p["ml2_e_b2"] @ W1a_5)
    x3, _, _, _, _ = _layer("ml5", x2b, row, col2d, eac5, uu, p,
                            cnt_inv, pos, 23, 10, 23, 32, zeros32)
    return x3.reshape(nB, 1, -1)


# speedup vs baseline: 5.5829x; 2.4272x over previous
import functools

import jax
import jax.numpy as jnp
from jax import lax
from jax.experimental import pallas as pl
from jax.experimental.pallas import tpu as pltpu
from jax.experimental.pallas import tpu_sc as plsc

B, C = 50, 1000
BE = 4000

NC, NS = 2, 16
NW = NC * NS
GU = 128
G = 5
KB = GU * G
NPAD = 50048

KBG = 2000
WRG = 32
WCG = 8


def _sc_scatter_call(payload, col2d, zeros):
    E, Wp = payload.shape
    npad = NPAD
    units = E // KB
    jmax = (units + NW - 1) // NW
    rows_per_sub = npad // NS
    chunks = []
    off = 0
    while off < rows_per_sub:
        sz = min(KB, rows_per_sub - off)
        chunks.append((off, sz))
        off += sz
    mesh = plsc.VectorSubcoreMesh(core_axis_name="c", subcore_axis_name="s")

    @functools.partial(
        pl.kernel,
        out_type=jax.ShapeDtypeStruct((NC * npad, Wp), jnp.float32),
        mesh=mesh,
        scratch_types=[
            pltpu.VMEM((G, GU), jnp.int32),
            pltpu.VMEM((KB, Wp), jnp.float32),
            pltpu.VMEM_SHARED((npad, Wp), jnp.float32),
        ],
        compiler_params=pltpu.CompilerParams(use_tc_tiling_on_sc=False),
    )
    def k(pay_hbm, col_hbm, zero_hbm, out_hbm, colbuf, paybuf, acc):
        c = lax.axis_index("c")
        s = lax.axis_index("s")
        w = s * NC + c
        rs = s * rows_per_sub
        pltpu.sync_copy(zero_hbm, paybuf)
        for off, sz in chunks:
            pltpu.sync_copy(paybuf.at[pl.ds(0, sz)],
                            acc.at[pl.ds(rs + off, sz)])
        plsc.subcore_barrier()

        def body(j, carry):
            u = w + NW * j

            @pl.when(u < units)
            def _():
                pltpu.sync_copy(col_hbm.at[u], colbuf)
                pltpu.sync_copy(pay_hbm.at[pl.ds(u * KB, KB)], paybuf)
                for g in range(G):
                    pltpu.sync_copy(paybuf.at[pl.ds(g * GU, GU)],
                                    acc.at[colbuf.at[g]], add=True)

            return carry

        lax.fori_loop(0, jmax, body, 0)
        plsc.subcore_barrier()
        for off, sz in chunks:
            pltpu.sync_copy(acc.at[pl.ds(rs + off, sz)],
                            paybuf.at[pl.ds(0, sz)])
            pltpu.sync_copy(paybuf.at[pl.ds(0, sz)],
                            out_hbm.at[pl.ds(c * npad + rs + off, sz)])

    return k(payload, col2d, zeros)


def _sc_gather_call(tabr, tabc, row, col):
    E = row.shape[0]
    WR = tabr.shape[1]
    WC = tabc.shape[1]
    units = E // KBG
    jmax = (units + NW - 1) // NW
    mesh = plsc.VectorSubcoreMesh(core_axis_name="c", subcore_axis_name="s")

    @functools.partial(
        pl.kernel,
        out_type=[
            jax.ShapeDtypeStruct((E, WR), jnp.float32),
            jax.ShapeDtypeStruct((E, WC), jnp.float32),
        ],
        mesh=mesh,
        scratch_types=[
            pltpu.VMEM((KBG,), jnp.int32),
            pltpu.VMEM((KBG,), jnp.int32),
            pltpu.VMEM((KBG, WR), jnp.float32),
            pltpu.VMEM((KBG, WC), jnp.float32),
            pltpu.SemaphoreType.DMA,
            pltpu.SemaphoreType.DMA,
        ],
        compiler_params=pltpu.CompilerParams(use_tc_tiling_on_sc=False),
    )
    def gk(tabr_hbm, tabc_hbm, row_hbm, col_hbm, outr_hbm, outc_hbm,
           rowbuf, colbuf, grbuf, gcbuf, sem1, sem2):
        c = lax.axis_index("c")
        s = lax.axis_index("s")
        w = s * NC + c

        def body(j, carry):
            u = w + NW * j

            @pl.when(u < units)
            def _():
                base = u * KBG
                pltpu.sync_copy(row_hbm.at[pl.ds(base, KBG)], rowbuf)
                pltpu.sync_copy(col_hbm.at[pl.ds(base, KBG)], colbuf)
                d1 = pltpu.async_copy(tabr_hbm.at[rowbuf], grbuf, sem1)
                d2 = pltpu.async_copy(tabc_hbm.at[colbuf], gcbuf, sem2)
                d1.wait()
                d2.wait()
                pltpu.sync_copy(grbuf, outr_hbm.at[pl.ds(base, KBG)])
                pltpu.sync_copy(gcbuf, outc_hbm.at[pl.ds(base, KBG)])

            return carry

        lax.fori_loop(0, jmax, body, 0)

    return gk(tabr, tabc, row, col)


def _edge_block_kernel(gr_ref, gc_ref, eac_ref, a_ref, pay_ref, *, Hn1, Wp,
                       ones_col):
    gr = gr_ref[...]
    gc = gc_ref[...]
    eac = eac_ref[...]
    a = a_ref[...]
    h = jnp.maximum(gr[:, :5] + gc[:, :5] + eac, 0.0)
    z = jnp.maximum(gr[:, 5:5 + Hn1] + jax.lax.dot(h, a), 0.0)
    pad = Wp - 5 - Hn1
    parts = [h, z]
    if ones_col:
        parts.append(jnp.ones((h.shape[0], 1), jnp.float32))
        pad -= 1
    if pad:
        parts.append(jnp.zeros((h.shape[0], pad), jnp.float32))
    pay_ref[...] = jnp.concatenate(parts, axis=1)


def _edge_compute(gr, gc, eac, a, Hn1, Wp, ones_col):
    E = gr.shape[0]
    grid = (E // BE,)
    return pl.pallas_call(
        functools.partial(_edge_block_kernel, Hn1=Hn1, Wp=Wp,
                          ones_col=ones_col),
        grid=grid,
        in_specs=[
            pl.BlockSpec((BE, WRG), lambda i: (i, 0)),
            pl.BlockSpec((BE, WCG), lambda i: (i, 0)),
            pl.BlockSpec((BE, 5), lambda i: (i, 0)),
            pl.BlockSpec((5, Hn1), lambda i: (0, 0)),
        ],
        out_specs=pl.BlockSpec((BE, Wp), lambda i: (i, 0)),
        out_shape=jax.ShapeDtypeStruct((E, Wp), jnp.float32),
    )(gr, gc, eac, a)


def _mlp(h, p, name):
    h = jnp.maximum(h @ p[name + "_w1"] + p[name + "_b1"], 0.0)
    return h @ p[name + "_w2"] + p[name + "_b2"]


def _layer(prefix, x, row, col, col2d, eac, u, p, cnt_inv, pos, Fx, Fa, Hn1,
           Wp, zeros, ones_col=False):
    W1 = p[prefix + "_e_w1"]; b1 = p[prefix + "_e_b1"]
    W2e = p[prefix + "_e_w2"]; b2e = p[prefix + "_e_b2"]
    V1 = p[prefix + "_n1_w1"]; b1n = p[prefix + "_n1_b1"]
    V2 = p[prefix + "_n1_w2"]; b2n = p[prefix + "_n1_b2"]
    W1_r, W1_c, W1_u = W1[:Fx], W1[Fx:2 * Fx], W1[2 * Fx + Fa:]
    V1_x, V1_a = V1[:Fx], V1[Fx:]

    U1bc = jnp.repeat(u @ W1_u, C, axis=0)
    PRe = x @ W1_r + U1bc + b1
    PRn = x @ V1_x + (b1n + b2e @ V1_a)
    PCe = x @ W1_c
    A = W2e @ V1_a

    N = x.shape[0]
    tabr = jnp.concatenate(
        [PRe, PRn, jnp.zeros((N, WRG - 5 - Hn1), jnp.float32)], axis=1)
    tabc = jnp.concatenate(
        [PCe, jnp.zeros((N, WCG - 5), jnp.float32)], axis=1)
    gr, gc = _sc_gather_call(tabr, tabc, row, col)
    payload = _edge_compute(gr, gc, eac, A, Hn1, Wp, ones_col)
    h = payload[:, :5]

    parts = _sc_scatter_call(payload, col2d, zeros)
    S = parts[:N] + parts[NPAD:NPAD + N]
    Sh, Sz = S[:, :5], S[:, 5:5 + Hn1]
    if ones_col:
        cnt = S[:, 5 + Hn1:6 + Hn1]
        cnt_inv = 1.0 / jnp.maximum(cnt, 1.0)
        pos = jnp.where(cnt > 0, 1.0, 0.0)
    agg = (Sz * cnt_inv) @ V2 + b2n * pos
    edge_u = (Sh * cnt_inv) @ W2e + b2e * pos

    xu = jnp.repeat(u, C, axis=0)
    x_new = _mlp(jnp.concatenate([x, agg, xu], axis=1), p, prefix + "_n2")
    return x_new, h, edge_u, cnt_inv, pos


def _global_model(prefix, u, x_new, edge_u, p):
    xbar = x_new.reshape(B, C, -1).mean(axis=1)
    ebar = edge_u.reshape(B, C, -1).mean(axis=1)
    return _mlp(jnp.concatenate([u, xbar, ebar], axis=1), p, prefix + "_g")


def kernel(deepgo, x, u, edge, edge_attr, batch, params):
    p = params
    nB = deepgo.shape[0]
    N = x.shape[0]
    row, col = edge[0], edge[1]
    col2d = col.reshape(-1, G, GU)
    node_dpg = jnp.squeeze(deepgo, 1).reshape(C * nB, 1)
    uu = jnp.squeeze(u[:, :, 1:21], 1)

    zeros24 = jnp.zeros((KB, 24), jnp.float32)
    zeros32 = jnp.zeros((KB, 32), jnp.float32)

    W1a_1 = p["ml1_e_w1"][2 * 12:2 * 12 + 1]
    eac1 = edge_attr @ W1a_1
    x1, h1, eu1, cnt_inv, pos = _layer("ml1", x, row, col, col2d, eac1, uu, p,
                                       None, None, 12, 1, 15, 24, zeros24,
                                       ones_col=True)
    uu = _global_model("ml1", uu, x1, eu1, p)

    W1a_2 = p["ml2_e_w1"][2 * 20:2 * 20 + 10]
    eac2 = h1 @ (p["ml1_e_w2"] @ W1a_2) + (p["ml1_e_b2"] @ W1a_2)
    x2, h2, eu2, _, _ = _layer("ml2", x1, row, col, col2d, eac2, uu, p,
                               cnt_inv, pos, 20, 10, 20, 32, zeros32)
    uu = _global_model("ml2", uu, x2, eu2, p)

    x2b = jnp.concatenate([x2, node_dpg, x[:, 10:]], axis=1)
    W1a_5 = p["ml5_e_w1"][2 * 23:2 * 23 + 10]
    eac5 = h2 @ (p["ml2_e_w2"] @ W1a_5) + (p["ml2_e_b2"] @ W1a_5)
    x3, _, _, _, _ = _layer("ml5", x2b, row, col, col2d, eac5, uu, p,
                            cnt_inv, pos, 23, 10, 23, 32, zeros32)
    return x3.reshape(nB, 1, -1)


# speedup vs baseline: 8.2557x; 1.4787x over previous
import functools

import jax
import jax.numpy as jnp
from jax import lax
from jax.experimental import pallas as pl
from jax.experimental.pallas import tpu as pltpu
from jax.experimental.pallas import tpu_sc as plsc

B, C = 50, 1000
BE = 4000

NC, NS = 2, 16
NW = NC * NS
GU = 128
G = 5
KB = GU * G
NPAD = 50048

KBG = 2000
WRG = 32
WCG = 8

KF = 800
MATW = 160


def _sc_scatter_call(payload, col2d, zeros):
    E, Wp = payload.shape
    npad = NPAD
    units = E // KB
    jmax = (units + NW - 1) // NW
    rows_per_sub = npad // NS
    chunks = []
    off = 0
    while off < rows_per_sub:
        sz = min(KB, rows_per_sub - off)
        chunks.append((off, sz))
        off += sz
    mesh = plsc.VectorSubcoreMesh(core_axis_name="c", subcore_axis_name="s")

    @functools.partial(
        pl.kernel,
        out_type=jax.ShapeDtypeStruct((NC * npad, Wp), jnp.float32),
        mesh=mesh,
        scratch_types=[
            pltpu.VMEM((G, GU), jnp.int32),
            pltpu.VMEM((KB, Wp), jnp.float32),
            pltpu.VMEM_SHARED((npad, Wp), jnp.float32),
        ],
        compiler_params=pltpu.CompilerParams(use_tc_tiling_on_sc=False),
    )
    def k(pay_hbm, col_hbm, zero_hbm, out_hbm, colbuf, paybuf, acc):
        c = lax.axis_index("c")
        s = lax.axis_index("s")
        w = s * NC + c
        rs = s * rows_per_sub
        pltpu.sync_copy(zero_hbm, paybuf)
        for off, sz in chunks:
            pltpu.sync_copy(paybuf.at[pl.ds(0, sz)],
                            acc.at[pl.ds(rs + off, sz)])
        plsc.subcore_barrier()

        def body(j, carry):
            u = w + NW * j

            @pl.when(u < units)
            def _():
                pltpu.sync_copy(col_hbm.at[u], colbuf)
                pltpu.sync_copy(pay_hbm.at[pl.ds(u * KB, KB)], paybuf)
                for g in range(G):
                    pltpu.sync_copy(paybuf.at[pl.ds(g * GU, GU)],
                                    acc.at[colbuf.at[g]], add=True)

            return carry

        lax.fori_loop(0, jmax, body, 0)
        plsc.subcore_barrier()
        for off, sz in chunks:
            pltpu.sync_copy(acc.at[pl.ds(rs + off, sz)],
                            paybuf.at[pl.ds(0, sz)])
            pltpu.sync_copy(paybuf.at[pl.ds(0, sz)],
                            out_hbm.at[pl.ds(c * npad + rs + off, sz)])

    return k(payload, col2d, zeros)


def _sc_gather_call(tabr, tabc, row, col):
    E = row.shape[0]
    WR = tabr.shape[1]
    WC = tabc.shape[1]
    units = E // KBG
    jmax = (units + NW - 1) // NW
    mesh = plsc.VectorSubcoreMesh(core_axis_name="c", subcore_axis_name="s")

    @functools.partial(
        pl.kernel,
        out_type=[
            jax.ShapeDtypeStruct((E, WR), jnp.float32),
            jax.ShapeDtypeStruct((E, WC), jnp.float32),
        ],
        mesh=mesh,
        scratch_types=[
            pltpu.VMEM((KBG,), jnp.int32),
            pltpu.VMEM((KBG,), jnp.int32),
            pltpu.VMEM((KBG, WR), jnp.float32),
            pltpu.VMEM((KBG, WC), jnp.float32),
            pltpu.SemaphoreType.DMA,
            pltpu.SemaphoreType.DMA,
        ],
        compiler_params=pltpu.CompilerParams(use_tc_tiling_on_sc=False),
    )
    def gk(tabr_hbm, tabc_hbm, row_hbm, col_hbm, outr_hbm, outc_hbm,
           rowbuf, colbuf, grbuf, gcbuf, sem1, sem2):
        c = lax.axis_index("c")
        s = lax.axis_index("s")
        w = s * NC + c

        def body(j, carry):
            u = w + NW * j

            @pl.when(u < units)
            def _():
                base = u * KBG
                pltpu.sync_copy(row_hbm.at[pl.ds(base, KBG)], rowbuf)
                pltpu.sync_copy(col_hbm.at[pl.ds(base, KBG)], colbuf)
                d1 = pltpu.async_copy(tabr_hbm.at[rowbuf], grbuf, sem1)
                d2 = pltpu.async_copy(tabc_hbm.at[colbuf], gcbuf, sem2)
                d1.wait()
                d2.wait()
                pltpu.sync_copy(grbuf, outr_hbm.at[pl.ds(base, KBG)])
                pltpu.sync_copy(gcbuf, outc_hbm.at[pl.ds(base, KBG)])

            return carry

        lax.fori_loop(0, jmax, body, 0)

    return gk(tabr, tabc, row, col)


def _sc_edge_call(tabr, tabc, row, col, prev, mat, zeros, Hn1, Wp, prev_is_h,
                  ones_col, emit_h):
    E = row.shape[0]
    units = E // KF
    jmax = (units + NW - 1) // NW
    ngrp = KF // 16
    moff = 32 if prev_is_h else 16
    mesh = plsc.VectorSubcoreMesh(core_axis_name="c", subcore_axis_name="s")
    hout_shape = (E, 8) if emit_h else (8, 8)

    @functools.partial(
        pl.kernel,
        out_type=[
            jax.ShapeDtypeStruct((E, Wp), jnp.float32),
            jax.ShapeDtypeStruct(hout_shape, jnp.float32),
        ],
        mesh=mesh,
        scratch_types=[
            pltpu.VMEM((KF,), jnp.int32),
            pltpu.VMEM((KF,), jnp.int32),
            pltpu.VMEM((KF, WRG), jnp.float32),
            pltpu.VMEM((KF, WCG), jnp.float32),
            pltpu.VMEM((KF, 8) if prev_is_h else (KF,), jnp.float32),
            pltpu.VMEM((KF, Wp), jnp.float32),
            pltpu.VMEM((KF, 8), jnp.float32),
            pltpu.VMEM((MATW,), jnp.float32),
            pltpu.SemaphoreType.DMA,
            pltpu.SemaphoreType.DMA,
        ],
        compiler_params=pltpu.CompilerParams(use_tc_tiling_on_sc=False,
                                             needs_layout_passes=False),
    )
    def gk(tabr_hbm, tabc_hbm, row_hbm, col_hbm, prev_hbm, mat_hbm, zero_hbm,
           pay_out, hout_out, rowbuf, colbuf, grbuf, gcbuf, pvbuf, paybuf,
           houtbuf, mat_s, sem1, sem2):
        c = lax.axis_index("c")
        s = lax.axis_index("s")
        w = s * NC + c
        pltpu.sync_copy(mat_hbm, mat_s)
        pltpu.sync_copy(zero_hbm, paybuf)
        iota16 = lax.iota(jnp.int32, 16)
        mv = [mat_s[pl.ds(16 * i, 16)] for i in range(MATW // 16)]

        def ms(i):
            return mv[i // 16][i % 16]

        def cf(v):
            return jnp.full((16,), v, jnp.int32)

        def body(j, carry):
            u = w + NW * j

            @pl.when(u < units)
            def _():
                base = u * KF
                pltpu.sync_copy(row_hbm.at[pl.ds(base, KF)], rowbuf)
                pltpu.sync_copy(col_hbm.at[pl.ds(base, KF)], colbuf)
                pltpu.sync_copy(prev_hbm.at[pl.ds(base, KF)], pvbuf)
                d1 = pltpu.async_copy(tabr_hbm.at[rowbuf], grbuf, sem1)
                d2 = pltpu.async_copy(tabc_hbm.at[colbuf], gcbuf, sem2)
                d1.wait()
                d2.wait()

                def grp(g, cr):
                    rid = g * 16 + iota16
                    if prev_is_h:
                        hp = [plsc.load_gather(pvbuf, [rid, cf(k)])
                              for k in range(5)]
                        contrib = [sum(hp[k] * ms(k * 5 + f)
                                       for k in range(5)) for f in range(5)]
                    else:
                        ea = plsc.load_gather(pvbuf, [rid])
                        contrib = [ea * ms(f) for f in range(5)]
                    h = []
                    for f in range(5):
                        pre = (plsc.load_gather(grbuf, [rid, cf(f)])
                               + plsc.load_gather(gcbuf, [rid, cf(f)])
                               + contrib[f])
                        h.append(jnp.maximum(pre, 0.0))
                    for f in range(5):
                        plsc.store_scatter(paybuf, [rid, cf(f)], h[f])
                        if emit_h:
                            plsc.store_scatter(houtbuf, [rid, cf(f)], h[f])
                    for jz in range(Hn1):
                        zpre = plsc.load_gather(grbuf, [rid, cf(5 + jz)])
                        zpre = zpre + sum(
                            h[k] * ms(moff + k * Hn1 + jz)
                            for k in range(5))
                        plsc.store_scatter(paybuf, [rid, cf(5 + jz)],
                                           jnp.maximum(zpre, 0.0))
                    if ones_col:
                        plsc.store_scatter(paybuf, [rid, cf(5 + Hn1)],
                                           jnp.full((16,), 1.0, jnp.float32))
                    return cr

                lax.fori_loop(0, ngrp, grp, 0)
                pltpu.sync_copy(paybuf, pay_out.at[pl.ds(base, KF)])
                if emit_h:
                    pltpu.sync_copy(houtbuf, hout_out.at[pl.ds(base, KF)])

            return carry

        lax.fori_loop(0, jmax, body, 0)

    return gk(tabr, tabc, row, col, prev, mat, zeros)


def _edge_block_kernel(gr_ref, gc_ref, eac_ref, a_ref, pay_ref, *, Hn1, Wp,
                       ones_col):
    gr = gr_ref[...]
    gc = gc_ref[...]
    eac = eac_ref[...]
    a = a_ref[...]
    h = jnp.maximum(gr[:, :5] + gc[:, :5] + eac, 0.0)
    z = jnp.maximum(gr[:, 5:5 + Hn1] + jax.lax.dot(h, a), 0.0)
    pad = Wp - 5 - Hn1
    parts = [h, z]
    if ones_col:
        parts.append(jnp.ones((h.shape[0], 1), jnp.float32))
        pad -= 1
    if pad:
        parts.append(jnp.zeros((h.shape[0], pad), jnp.float32))
    pay_ref[...] = jnp.concatenate(parts, axis=1)


def _edge_compute(gr, gc, eac, a, Hn1, Wp, ones_col):
    E = gr.shape[0]
    grid = (E // BE,)
    return pl.pallas_call(
        functools.partial(_edge_block_kernel, Hn1=Hn1, Wp=Wp,
                          ones_col=ones_col),
        grid=grid,
        in_specs=[
            pl.BlockSpec((BE, WRG), lambda i: (i, 0)),
            pl.BlockSpec((BE, WCG), lambda i: (i, 0)),
            pl.BlockSpec((BE, 5), lambda i: (i, 0)),
            pl.BlockSpec((5, Hn1), lambda i: (0, 0)),
        ],
        out_specs=pl.BlockSpec((BE, Wp), lambda i: (i, 0)),
        out_shape=jax.ShapeDtypeStruct((E, Wp), jnp.float32),
    )(gr, gc, eac, a)


def _combine_block(p_ref, q_ref, o_ref):
    o_ref[...] = p_ref[...] + q_ref[...]


def _combine_halves(parts, N):
    Wp = parts.shape[1]
    BN = NPAD // 8
    S = pl.pallas_call(
        _combine_block,
        grid=(NPAD // BN,),
        in_specs=[pl.BlockSpec((BN, Wp), lambda i: (i, 0)),
                  pl.BlockSpec((BN, Wp), lambda i: (i, 0))],
        out_specs=pl.BlockSpec((BN, Wp), lambda i: (i, 0)),
        out_shape=jax.ShapeDtypeStruct((NPAD, Wp), jnp.float32),
    )(parts[:NPAD], parts[NPAD:])
    return S[:N]


def _mlp(h, p, name):
    h = jnp.maximum(h @ p[name + "_w1"] + p[name + "_b1"], 0.0)
    return h @ p[name + "_w2"] + p[name + "_b2"]


def _layer(prefix, x, row, col, col2d, prev, mhead, pre_const, u, p, cnt_inv,
           pos, Fx, Fa, Hn1, Wp, zeros, zeros_kf, prev_is_h,
           ones_col=False, emit_h=True):
    W1 = p[prefix + "_e_w1"]; b1 = p[prefix + "_e_b1"]
    W2e = p[prefix + "_e_w2"]; b2e = p[prefix + "_e_b2"]
    V1 = p[prefix + "_n1_w1"]; b1n = p[prefix + "_n1_b1"]
    V2 = p[prefix + "_n1_w2"]; b2n = p[prefix + "_n1_b2"]
    W1_r, W1_c, W1_u = W1[:Fx], W1[Fx:2 * Fx], W1[2 * Fx + Fa:]
    V1_x, V1_a = V1[:Fx], V1[Fx:]

    U1bc = jnp.repeat(u @ W1_u, C, axis=0)
    PRe = x @ W1_r + U1bc + (b1 + pre_const)
    PRn = x @ V1_x + (b1n + b2e @ V1_a)
    PCe = x @ W1_c
    A = W2e @ V1_a

    moff = 32 if prev_is_h else 16
    mat = jnp.zeros((MATW,), jnp.float32)
    mat = mat.at[:mhead.size].set(mhead.reshape(-1))
    mat = mat.at[moff:moff + 5 * Hn1].set(A.reshape(-1))

    N = x.shape[0]
    tabr = jnp.concatenate(
        [PRe, PRn, jnp.zeros((N, WRG - 5 - Hn1), jnp.float32)], axis=1)
    tabc = jnp.concatenate(
        [PCe, jnp.zeros((N, WCG - 5), jnp.float32)], axis=1)
    payload, hout = _sc_edge_call(tabr, tabc, row, col, prev, mat, zeros_kf,
                                  Hn1, Wp, prev_is_h, ones_col, emit_h)

    parts = _sc_scatter_call(payload, col2d, zeros)
    S = _combine_halves(parts, N)
    Sh, Sz = S[:, :5], S[:, 5:5 + Hn1]
    if ones_col:
        cnt = S[:, 5 + Hn1:6 + Hn1]
        cnt_inv = 1.0 / jnp.maximum(cnt, 1.0)
        pos = jnp.where(cnt > 0, 1.0, 0.0)
    agg = (Sz * cnt_inv) @ V2 + b2n * pos
    edge_u = (Sh * cnt_inv) @ W2e + b2e * pos

    xu = jnp.repeat(u, C, axis=0)
    x_new = _mlp(jnp.concatenate([x, agg, xu], axis=1), p, prefix + "_n2")
    return x_new, hout, edge_u, cnt_inv, pos


def _global_model(prefix, u, x_new, edge_u, p):
    xbar = x_new.reshape(B, C, -1).mean(axis=1)
    ebar = edge_u.reshape(B, C, -1).mean(axis=1)
    return _mlp(jnp.concatenate([u, xbar, ebar], axis=1), p, prefix + "_g")


def kernel(deepgo, x, u, edge, edge_attr, batch, params):
    p = params
    nB = deepgo.shape[0]
    N = x.shape[0]
    row, col = edge[0], edge[1]
    col2d = col.reshape(-1, G, GU)
    node_dpg = jnp.squeeze(deepgo, 1).reshape(C * nB, 1)
    uu = jnp.squeeze(u[:, :, 1:21], 1)

    zeros24 = jnp.zeros((KB, 24), jnp.float32)
    zeros32 = jnp.zeros((KB, 32), jnp.float32)
    zkf24 = jnp.zeros((KF, 24), jnp.float32)
    zkf32 = jnp.zeros((KF, 32), jnp.float32)
    zero5 = jnp.zeros((5,), jnp.float32)

    W1a_1 = p["ml1_e_w1"][2 * 12]
    ea1d = jnp.squeeze(edge_attr, 1)
    x1, h1, eu1, cnt_inv, pos = _layer(
        "ml1", x, row, col, col2d, ea1d, W1a_1, zero5, uu, p,
        None, None, 12, 1, 15, 24, zeros24, zkf24, False, ones_col=True)
    uu = _global_model("ml1", uu, x1, eu1, p)

    W1a_2 = p["ml2_e_w1"][2 * 20:2 * 20 + 10]
    M2 = p["ml1_e_w2"] @ W1a_2
    c2 = p["ml1_e_b2"] @ W1a_2
    x2, h2, eu2, _, _ = _layer(
        "ml2", x1, row, col, col2d, h1, M2, c2, uu, p,
        cnt_inv, pos, 20, 10, 20, 32, zeros32, zkf32, True)
    uu = _global_model("ml2", uu, x2, eu2, p)

    x2b = jnp.concatenate([x2, node_dpg, x[:, 10:]], axis=1)
    W1a_5 = p["ml5_e_w1"][2 * 23:2 * 23 + 10]
    M5 = p["ml2_e_w2"] @ W1a_5
    c5 = p["ml2_e_b2"] @ W1a_5
    x3, _, _, _, _ = _layer(
        "ml5", x2b, row, col, col2d, h2, M5, c5, uu, p,
        cnt_inv, pos, 23, 10, 23, 32, zeros32, zkf32, True, emit_h=False)
    return x3.reshape(nB, 1, -1)


# speedup vs baseline: 8.2889x; 1.0040x over previous
import functools

import jax
import jax.numpy as jnp
from jax import lax
from jax.experimental import pallas as pl
from jax.experimental.pallas import tpu as pltpu
from jax.experimental.pallas import tpu_sc as plsc

B, C = 50, 1000
BE = 4000

NC, NS = 2, 16
NW = NC * NS
GU = 128
G = 5
KB = GU * G
NPAD = 50048

KBG = 2000
WRG = 32
WCG = 8

KF = 800
MATW = 160


def _sc_scatter_call(payload, col2d, zeros):
    E, Wp = payload.shape
    npad = NPAD
    units = E // KB
    jmax = (units + NW - 1) // NW
    rows_per_sub = npad // NS
    chunks = []
    off = 0
    while off < rows_per_sub:
        sz = min(KB, rows_per_sub - off)
        chunks.append((off, sz))
        off += sz
    mesh = plsc.VectorSubcoreMesh(core_axis_name="c", subcore_axis_name="s")

    @functools.partial(
        pl.kernel,
        out_type=jax.ShapeDtypeStruct((NC * npad, Wp), jnp.float32),
        mesh=mesh,
        scratch_types=[
            pltpu.VMEM((G, GU), jnp.int32),
            pltpu.VMEM((KB, Wp), jnp.float32),
            pltpu.VMEM_SHARED((npad, Wp), jnp.float32),
        ],
        compiler_params=pltpu.CompilerParams(use_tc_tiling_on_sc=False),
    )
    def k(pay_hbm, col_hbm, zero_hbm, out_hbm, colbuf, paybuf, acc):
        c = lax.axis_index("c")
        s = lax.axis_index("s")
        w = s * NC + c
        rs = s * rows_per_sub
        pltpu.sync_copy(zero_hbm, paybuf)
        for off, sz in chunks:
            pltpu.sync_copy(paybuf.at[pl.ds(0, sz)],
                            acc.at[pl.ds(rs + off, sz)])
        plsc.subcore_barrier()

        def body(j, carry):
            u = w + NW * j

            @pl.when(u < units)
            def _():
                pltpu.sync_copy(col_hbm.at[u], colbuf)
                pltpu.sync_copy(pay_hbm.at[pl.ds(u * KB, KB)], paybuf)
                for g in range(G):
                    pltpu.sync_copy(paybuf.at[pl.ds(g * GU, GU)],
                                    acc.at[colbuf.at[g]], add=True)

            return carry

        lax.fori_loop(0, jmax, body, 0)
        plsc.subcore_barrier()
        for off, sz in chunks:
            pltpu.sync_copy(acc.at[pl.ds(rs + off, sz)],
                            paybuf.at[pl.ds(0, sz)])
            pltpu.sync_copy(paybuf.at[pl.ds(0, sz)],
                            out_hbm.at[pl.ds(c * npad + rs + off, sz)])

    return k(payload, col2d, zeros)


def _sc_gather_call(tabr, tabc, row, col):
    E = row.shape[0]
    WR = tabr.shape[1]
    WC = tabc.shape[1]
    units = E // KBG
    jmax = (units + NW - 1) // NW
    mesh = plsc.VectorSubcoreMesh(core_axis_name="c", subcore_axis_name="s")

    @functools.partial(
        pl.kernel,
        out_type=[
            jax.ShapeDtypeStruct((E, WR), jnp.float32),
            jax.ShapeDtypeStruct((E, WC), jnp.float32),
        ],
        mesh=mesh,
        scratch_types=[
            pltpu.VMEM((KBG,), jnp.int32),
            pltpu.VMEM((KBG,), jnp.int32),
            pltpu.VMEM((KBG, WR), jnp.float32),
            pltpu.VMEM((KBG, WC), jnp.float32),
            pltpu.SemaphoreType.DMA,
            pltpu.SemaphoreType.DMA,
        ],
        compiler_params=pltpu.CompilerParams(use_tc_tiling_on_sc=False),
    )
    def gk(tabr_hbm, tabc_hbm, row_hbm, col_hbm, outr_hbm, outc_hbm,
           rowbuf, colbuf, grbuf, gcbuf, sem1, sem2):
        c = lax.axis_index("c")
        s = lax.axis_index("s")
        w = s * NC + c

        def body(j, carry):
            u = w + NW * j

            @pl.when(u < units)
            def _():
                base = u * KBG
                pltpu.sync_copy(row_hbm.at[pl.ds(base, KBG)], rowbuf)
                pltpu.sync_copy(col_hbm.at[pl.ds(base, KBG)], colbuf)
                d1 = pltpu.async_copy(tabr_hbm.at[rowbuf], grbuf, sem1)
                d2 = pltpu.async_copy(tabc_hbm.at[colbuf], gcbuf, sem2)
                d1.wait()
                d2.wait()
                pltpu.sync_copy(grbuf, outr_hbm.at[pl.ds(base, KBG)])
                pltpu.sync_copy(gcbuf, outc_hbm.at[pl.ds(base, KBG)])

            return carry

        lax.fori_loop(0, jmax, body, 0)

    return gk(tabr, tabc, row, col)


def _sc_edge_call(tabr, tabc, row, col, prev, mat, zeros, Hn1, Wp, prev_is_h,
                  ones_col, emit_h):
    E = row.shape[0]
    units = E // KF
    jmax = (units + NW - 1) // NW
    ngrp = KF // 16
    moff = 32 if prev_is_h else 16
    mesh = plsc.VectorSubcoreMesh(core_axis_name="c", subcore_axis_name="s")
    hout_shape = (E, 8) if emit_h else (8, 8)

    @functools.partial(
        pl.kernel,
        out_type=[
            jax.ShapeDtypeStruct((E, Wp), jnp.float32),
            jax.ShapeDtypeStruct(hout_shape, jnp.float32),
        ],
        mesh=mesh,
        scratch_types=[
            pltpu.VMEM((KF,), jnp.int32),
            pltpu.VMEM((KF,), jnp.int32),
            pltpu.VMEM((KF, WRG), jnp.float32),
            pltpu.VMEM((KF, WCG), jnp.float32),
            pltpu.VMEM((KF, 8) if prev_is_h else (KF,), jnp.float32),
            pltpu.VMEM((KF, Wp), jnp.float32),
            pltpu.VMEM((KF, 8), jnp.float32),
            pltpu.VMEM((MATW,), jnp.float32),
            pltpu.SemaphoreType.DMA,
            pltpu.SemaphoreType.DMA,
        ],
        compiler_params=pltpu.CompilerParams(use_tc_tiling_on_sc=False,
                                             needs_layout_passes=False),
    )
    def gk(tabr_hbm, tabc_hbm, row_hbm, col_hbm, prev_hbm, mat_hbm, zero_hbm,
           pay_out, hout_out, rowbuf, colbuf, grbuf, gcbuf, pvbuf, paybuf,
           houtbuf, mat_s, sem1, sem2):
        c = lax.axis_index("c")
        s = lax.axis_index("s")
        w = s * NC + c
        pltpu.sync_copy(mat_hbm, mat_s)
        pltpu.sync_copy(zero_hbm, paybuf)
        iota16 = lax.iota(jnp.int32, 16)
        mv = [mat_s[pl.ds(16 * i, 16)] for i in range(MATW // 16)]
        ones16 = jnp.full((16,), 1.0, jnp.float32)

        def ms(i):
            return mv[i // 16][i % 16]

        if prev_is_h:
            bM = [ones16 * ms(i) for i in range(25)]
        else:
            bM = [ones16 * ms(i) for i in range(5)]
        bA = [ones16 * ms(moff + i) for i in range(5 * Hn1)]

        def cf(v):
            return jnp.full((16,), v, jnp.int32)

        def body(j, carry):
            u = w + NW * j

            @pl.when(u < units)
            def _():
                base = u * KF
                pltpu.sync_copy(row_hbm.at[pl.ds(base, KF)], rowbuf)
                pltpu.sync_copy(col_hbm.at[pl.ds(base, KF)], colbuf)
                pltpu.sync_copy(prev_hbm.at[pl.ds(base, KF)], pvbuf)
                d1 = pltpu.async_copy(tabr_hbm.at[rowbuf], grbuf, sem1)
                d2 = pltpu.async_copy(tabc_hbm.at[colbuf], gcbuf, sem2)
                d1.wait()
                d2.wait()

                def grp(g, cr):
                    rid = g * 16 + iota16
                    if prev_is_h:
                        hp = [plsc.load_gather(pvbuf, [rid, cf(k)])
                              for k in range(5)]
                        contrib = [sum(hp[k] * bM[k * 5 + f]
                                       for k in range(5)) for f in range(5)]
                    else:
                        ea = plsc.load_gather(pvbuf, [rid])
                        contrib = [ea * bM[f] for f in range(5)]
                    h = []
                    for f in range(5):
                        pre = (plsc.load_gather(grbuf, [rid, cf(f)])
                               + plsc.load_gather(gcbuf, [rid, cf(f)])
                               + contrib[f])
                        h.append(jnp.maximum(pre, 0.0))
                    for f in range(5):
                        plsc.store_scatter(paybuf, [rid, cf(f)], h[f])
                        if emit_h:
                            plsc.store_scatter(houtbuf, [rid, cf(f)], h[f])
                    for jz in range(Hn1):
                        zpre = plsc.load_gather(grbuf, [rid, cf(5 + jz)])
                        zpre = zpre + sum(
                            h[k] * bA[k * Hn1 + jz]
                            for k in range(5))
                        plsc.store_scatter(paybuf, [rid, cf(5 + jz)],
                                           jnp.maximum(zpre, 0.0))
                    if ones_col:
                        plsc.store_scatter(paybuf, [rid, cf(5 + Hn1)],
                                           jnp.full((16,), 1.0, jnp.float32))
                    return cr

                lax.fori_loop(0, ngrp, grp, 0)
                pltpu.sync_copy(paybuf, pay_out.at[pl.ds(base, KF)])
                if emit_h:
                    pltpu.sync_copy(houtbuf, hout_out.at[pl.ds(base, KF)])

            return carry

        lax.fori_loop(0, jmax, body, 0)

    return gk(tabr, tabc, row, col, prev, mat, zeros)


def _edge_block_kernel(gr_ref, gc_ref, eac_ref, a_ref, pay_ref, *, Hn1, Wp,
                       ones_col):
    gr = gr_ref[...]
    gc = gc_ref[...]
    eac = eac_ref[...]
    a = a_ref[...]
    h = jnp.maximum(gr[:, :5] + gc[:, :5] + eac, 0.0)
    z = jnp.maximum(gr[:, 5:5 + Hn1] + jax.lax.dot(h, a), 0.0)
    pad = Wp - 5 - Hn1
    parts = [h, z]
    if ones_col:
        parts.append(jnp.ones((h.shape[0], 1), jnp.float32))
        pad -= 1
    if pad:
        parts.append(jnp.zeros((h.shape[0], pad), jnp.float32))
    pay_ref[...] = jnp.concatenate(parts, axis=1)


def _edge_compute(gr, gc, eac, a, Hn1, Wp, ones_col):
    E = gr.shape[0]
    grid = (E // BE,)
    return pl.pallas_call(
        functools.partial(_edge_block_kernel, Hn1=Hn1, Wp=Wp,
                          ones_col=ones_col),
        grid=grid,
        in_specs=[
            pl.BlockSpec((BE, WRG), lambda i: (i, 0)),
            pl.BlockSpec((BE, WCG), lambda i: (i, 0)),
            pl.BlockSpec((BE, 5), lambda i: (i, 0)),
            pl.BlockSpec((5, Hn1), lambda i: (0, 0)),
        ],
        out_specs=pl.BlockSpec((BE, Wp), lambda i: (i, 0)),
        out_shape=jax.ShapeDtypeStruct((E, Wp), jnp.float32),
    )(gr, gc, eac, a)


def _combine_block(p_ref, q_ref, o_ref):
    o_ref[...] = p_ref[...] + q_ref[...]


def _combine_halves(parts, N):
    Wp = parts.shape[1]
    BN = NPAD // 8
    S = pl.pallas_call(
        _combine_block,
        grid=(NPAD // BN,),
        in_specs=[pl.BlockSpec((BN, Wp), lambda i: (i, 0)),
                  pl.BlockSpec((BN, Wp), lambda i: (i, 0))],
        out_specs=pl.BlockSpec((BN, Wp), lambda i: (i, 0)),
        out_shape=jax.ShapeDtypeStruct((NPAD, Wp), jnp.float32),
    )(parts[:NPAD], parts[NPAD:])
    return S[:N]


def _mlp(h, p, name):
    h = jnp.maximum(h @ p[name + "_w1"] + p[name + "_b1"], 0.0)
    return h @ p[name + "_w2"] + p[name + "_b2"]


def _layer(prefix, x, row, col, col2d, prev, mhead, pre_const, u, p, cnt_inv,
           pos, Fx, Fa, Hn1, Wp, zeros, zeros_kf, prev_is_h,
           ones_col=False, emit_h=True):
    W1 = p[prefix + "_e_w1"]; b1 = p[prefix + "_e_b1"]
    W2e = p[prefix + "_e_w2"]; b2e = p[prefix + "_e_b2"]
    V1 = p[prefix + "_n1_w1"]; b1n = p[prefix + "_n1_b1"]
    V2 = p[prefix + "_n1_w2"]; b2n = p[prefix + "_n1_b2"]
    W1_r, W1_c, W1_u = W1[:Fx], W1[Fx:2 * Fx], W1[2 * Fx + Fa:]
    V1_x, V1_a = V1[:Fx], V1[Fx:]

    U1bc = jnp.repeat(u @ W1_u, C, axis=0)
    PRe = x @ W1_r + U1bc + (b1 + pre_const)
    PRn = x @ V1_x + (b1n + b2e @ V1_a)
    PCe = x @ W1_c
    A = W2e @ V1_a

    moff = 32 if prev_is_h else 16
    mat = jnp.zeros((MATW,), jnp.float32)
    mat = mat.at[:mhead.size].set(mhead.reshape(-1))
    mat = mat.at[moff:moff + 5 * Hn1].set(A.reshape(-1))

    N = x.shape[0]
    tabr = jnp.concatenate(
        [PRe, PRn, jnp.zeros((N, WRG - 5 - Hn1), jnp.float32)], axis=1)
    tabc = jnp.concatenate(
        [PCe, jnp.zeros((N, WCG - 5), jnp.float32)], axis=1)
    payload, hout = _sc_edge_call(tabr, tabc, row, col, prev, mat, zeros_kf,
                                  Hn1, Wp, prev_is_h, ones_col, emit_h)

    parts = _sc_scatter_call(payload, col2d, zeros)
    S = _combine_halves(parts, N)
    Sh, Sz = S[:, :5], S[:, 5:5 + Hn1]
    if ones_col:
        cnt = S[:, 5 + Hn1:6 + Hn1]
        cnt_inv = 1.0 / jnp.maximum(cnt, 1.0)
        pos = jnp.where(cnt > 0, 1.0, 0.0)
    agg = (Sz * cnt_inv) @ V2 + b2n * pos
    edge_u = (Sh * cnt_inv) @ W2e + b2e * pos

    xu = jnp.repeat(u, C, axis=0)
    x_new = _mlp(jnp.concatenate([x, agg, xu], axis=1), p, prefix + "_n2")
    return x_new, hout, edge_u, cnt_inv, pos


def _global_model(prefix, u, x_new, edge_u, p):
    xbar = x_new.reshape(B, C, -1).mean(axis=1)
    ebar = edge_u.reshape(B, C, -1).mean(axis=1)
    return _mlp(jnp.concatenate([u, xbar, ebar], axis=1), p, prefix + "_g")


def kernel(deepgo, x, u, edge, edge_attr, batch, params):
    p = params
    nB = deepgo.shape[0]
    N = x.shape[0]
    row, col = edge[0], edge[1]
    col2d = col.reshape(-1, G, GU)
    node_dpg = jnp.squeeze(deepgo, 1).reshape(C * nB, 1)
    uu = jnp.squeeze(u[:, :, 1:21], 1)

    zeros24 = jnp.zeros((KB, 24), jnp.float32)
    zeros32 = jnp.zeros((KB, 32), jnp.float32)
    zkf24 = jnp.zeros((KF, 24), jnp.float32)
    zkf32 = jnp.zeros((KF, 32), jnp.float32)
    zero5 = jnp.zeros((5,), jnp.float32)

    W1a_1 = p["ml1_e_w1"][2 * 12]
    ea1d = jnp.squeeze(edge_attr, 1)
    x1, h1, eu1, cnt_inv, pos = _layer(
        "ml1", x, row, col, col2d, ea1d, W1a_1, zero5, uu, p,
        None, None, 12, 1, 15, 24, zeros24, zkf24, False, ones_col=True)
    uu = _global_model("ml1", uu, x1, eu1, p)

    W1a_2 = p["ml2_e_w1"][2 * 20:2 * 20 + 10]
    M2 = p["ml1_e_w2"] @ W1a_2
    c2 = p["ml1_e_b2"] @ W1a_2
    x2, h2, eu2, _, _ = _layer(
        "ml2", x1, row, col, col2d, h1, M2, c2, uu, p,
        cnt_inv, pos, 20, 10, 20, 32, zeros32, zkf32, True)
    uu = _global_model("ml2", uu, x2, eu2, p)

    x2b = jnp.concatenate([x2, node_dpg, x[:, 10:]], axis=1)
    W1a_5 = p["ml5_e_w1"][2 * 23:2 * 23 + 10]
    M5 = p["ml2_e_w2"] @ W1a_5
    c5 = p["ml2_e_b2"] @ W1a_5
    x3, _, _, _, _ = _layer(
        "ml5", x2b, row, col, col2d, h2, M5, c5, uu, p,
        cnt_inv, pos, 23, 10, 23, 32, zeros32, zkf32, True, emit_h=False)
    return x3.reshape(nB, 1, -1)


# speedup vs baseline: 8.2929x; 1.0005x over previous
import functools

import jax
import jax.numpy as jnp
from jax import lax
from jax.experimental import pallas as pl
from jax.experimental.pallas import tpu as pltpu
from jax.experimental.pallas import tpu_sc as plsc

B, C = 50, 1000

NC, NS = 2, 16
NW = NC * NS
GU = 128
G = 5
KB = GU * G
NPAD = 50048

WRG = 32
WCG = 8

KF = 800
MATW = 160


def _sc_scatter_call(payload, col2d, zeros):
    E, Wp = payload.shape
    npad = NPAD
    units = E // KB
    jmax = (units + NW - 1) // NW
    rows_per_sub = npad // NS
    chunks = []
    off = 0
    while off < rows_per_sub:
        sz = min(KB, rows_per_sub - off)
        chunks.append((off, sz))
        off += sz
    mesh = plsc.VectorSubcoreMesh(core_axis_name="c", subcore_axis_name="s")

    @functools.partial(
        pl.kernel,
        out_type=jax.ShapeDtypeStruct((NC * npad, Wp), jnp.float32),
        mesh=mesh,
        scratch_types=[
            pltpu.VMEM((G, GU), jnp.int32),
            pltpu.VMEM((KB, Wp), jnp.float32),
            pltpu.VMEM_SHARED((npad, Wp), jnp.float32),
        ],
        compiler_params=pltpu.CompilerParams(use_tc_tiling_on_sc=False),
    )
    def k(pay_hbm, col_hbm, zero_hbm, out_hbm, colbuf, paybuf, acc):
        c = lax.axis_index("c")
        s = lax.axis_index("s")
        w = s * NC + c
        rs = s * rows_per_sub
        pltpu.sync_copy(zero_hbm, paybuf)
        for off, sz in chunks:
            pltpu.sync_copy(paybuf.at[pl.ds(0, sz)],
                            acc.at[pl.ds(rs + off, sz)])
        plsc.subcore_barrier()

        def body(j, carry):
            u = w + NW * j

            @pl.when(u < units)
            def _():
                pltpu.sync_copy(col_hbm.at[u], colbuf)
                pltpu.sync_copy(pay_hbm.at[pl.ds(u * KB, KB)], paybuf)
                for g in range(G):
                    pltpu.sync_copy(paybuf.at[pl.ds(g * GU, GU)],
                                    acc.at[colbuf.at[g]], add=True)

            return carry

        lax.fori_loop(0, jmax, body, 0)
        plsc.subcore_barrier()
        for off, sz in chunks:
            pltpu.sync_copy(acc.at[pl.ds(rs + off, sz)],
                            paybuf.at[pl.ds(0, sz)])
            pltpu.sync_copy(paybuf.at[pl.ds(0, sz)],
                            out_hbm.at[pl.ds(c * npad + rs + off, sz)])

    return k(payload, col2d, zeros)


def _sc_edge_call(tabr, tabc, row, col, prev, mat, zeros, Hn1, Wp, prev_is_h,
                  ones_col, emit_h):
    E = row.shape[0]
    units = E // KF
    jmax = (units + NW - 1) // NW
    ngrp = KF // 16
    moff = 32 if prev_is_h else 16
    mesh = plsc.VectorSubcoreMesh(core_axis_name="c", subcore_axis_name="s")
    hout_shape = (E, 8) if emit_h else (8, 8)

    @functools.partial(
        pl.kernel,
        out_type=[
            jax.ShapeDtypeStruct((E, Wp), jnp.float32),
            jax.ShapeDtypeStruct(hout_shape, jnp.float32),
        ],
        mesh=mesh,
        scratch_types=[
            pltpu.VMEM((KF,), jnp.int32),
            pltpu.VMEM((KF,), jnp.int32),
            pltpu.VMEM((KF, WRG), jnp.float32),
            pltpu.VMEM((KF, WCG), jnp.float32),
            pltpu.VMEM((KF, 8) if prev_is_h else (KF,), jnp.float32),
            pltpu.VMEM((KF, Wp), jnp.float32),
            pltpu.VMEM((KF, 8), jnp.float32),
            pltpu.VMEM((MATW,), jnp.float32),
            pltpu.SemaphoreType.DMA,
            pltpu.SemaphoreType.DMA,
        ],
        compiler_params=pltpu.CompilerParams(use_tc_tiling_on_sc=False,
                                             needs_layout_passes=False),
    )
    def gk(tabr_hbm, tabc_hbm, row_hbm, col_hbm, prev_hbm, mat_hbm, zero_hbm,
           pay_out, hout_out, rowbuf, colbuf, grbuf, gcbuf, pvbuf, paybuf,
           houtbuf, mat_s, sem1, sem2):
        c = lax.axis_index("c")
        s = lax.axis_index("s")
        w = s * NC + c
        pltpu.sync_copy(mat_hbm, mat_s)
        pltpu.sync_copy(zero_hbm, paybuf)
        iota16 = lax.iota(jnp.int32, 16)
        mv = [mat_s[pl.ds(16 * i, 16)] for i in range(MATW // 16)]
        ones16 = jnp.full((16,), 1.0, jnp.float32)

        def ms(i):
            return mv[i // 16][i % 16]

        if prev_is_h:
            bM = [ones16 * ms(i) for i in range(25)]
        else:
            bM = [ones16 * ms(i) for i in range(5)]
        bA = [ones16 * ms(moff + i) for i in range(5 * Hn1)]

        def cf(v):
            return jnp.full((16,), v, jnp.int32)

        def body(j, carry):
            u = w + NW * j

            @pl.when(u < units)
            def _():
                base = u * KF
                pltpu.sync_copy(row_hbm.at[pl.ds(base, KF)], rowbuf)
                pltpu.sync_copy(col_hbm.at[pl.ds(base, KF)], colbuf)
                pltpu.sync_copy(prev_hbm.at[pl.ds(base, KF)], pvbuf)
                d1 = pltpu.async_copy(tabr_hbm.at[rowbuf], grbuf, sem1)
                d2 = pltpu.async_copy(tabc_hbm.at[colbuf], gcbuf, sem2)
                d1.wait()
                d2.wait()

                def grp(g, cr):
                    rid = g * 16 + iota16
                    if prev_is_h:
                        hp = [plsc.load_gather(pvbuf, [rid, cf(k)])
                              for k in range(5)]
                        contrib = [sum(hp[k] * bM[k * 5 + f]
                                       for k in range(5)) for f in range(5)]
                    else:
                        ea = plsc.load_gather(pvbuf, [rid])
                        contrib = [ea * bM[f] for f in range(5)]
                    h = []
                    for f in range(5):
                        pre = (plsc.load_gather(grbuf, [rid, cf(f)])
                               + plsc.load_gather(gcbuf, [rid, cf(f)])
                               + contrib[f])
                        h.append(jnp.maximum(pre, 0.0))
                    for f in range(5):
                        plsc.store_scatter(paybuf, [rid, cf(f)], h[f])
                        if emit_h:
                            plsc.store_scatter(houtbuf, [rid, cf(f)], h[f])
                    for jz in range(Hn1):
                        zpre = plsc.load_gather(grbuf, [rid, cf(5 + jz)])
                        zpre = zpre + sum(
                            h[k] * bA[k * Hn1 + jz]
                            for k in range(5))
                        plsc.store_scatter(paybuf, [rid, cf(5 + jz)],
                                           jnp.maximum(zpre, 0.0))
                    if ones_col:
                        plsc.store_scatter(paybuf, [rid, cf(5 + Hn1)],
                                           jnp.full((16,), 1.0, jnp.float32))
                    return cr

                lax.fori_loop(0, ngrp, grp, 0)
                pltpu.sync_copy(paybuf, pay_out.at[pl.ds(base, KF)])
                if emit_h:
                    pltpu.sync_copy(houtbuf, hout_out.at[pl.ds(base, KF)])

            return carry

        lax.fori_loop(0, jmax, body, 0)

    return gk(tabr, tabc, row, col, prev, mat, zeros)


def _combine_block(p_ref, q_ref, o_ref):
    o_ref[...] = p_ref[...] + q_ref[...]


def _combine_halves(parts, N):
    Wp = parts.shape[1]
    BN = NPAD // 8
    S = pl.pallas_call(
        _combine_block,
        grid=(NPAD // BN,),
        in_specs=[pl.BlockSpec((BN, Wp), lambda i: (i, 0)),
                  pl.BlockSpec((BN, Wp), lambda i: (i, 0))],
        out_specs=pl.BlockSpec((BN, Wp), lambda i: (i, 0)),
        out_shape=jax.ShapeDtypeStruct((NPAD, Wp), jnp.float32),
    )(parts[:NPAD], parts[NPAD:])
    return S[:N]


def _mlp(h, p, name):
    h = jnp.maximum(h @ p[name + "_w1"] + p[name + "_b1"], 0.0)
    return h @ p[name + "_w2"] + p[name + "_b2"]


def _layer(prefix, x, row, col, col2d, prev, mhead, pre_const, u, p, cnt_inv,
           pos, Fx, Fa, Hn1, Wp, zeros, zeros_kf, prev_is_h,
           ones_col=False, emit_h=True):
    W1 = p[prefix + "_e_w1"]; b1 = p[prefix + "_e_b1"]
    W2e = p[prefix + "_e_w2"]; b2e = p[prefix + "_e_b2"]
    V1 = p[prefix + "_n1_w1"]; b1n = p[prefix + "_n1_b1"]
    V2 = p[prefix + "_n1_w2"]; b2n = p[prefix + "_n1_b2"]
    W1_r, W1_c, W1_u = W1[:Fx], W1[Fx:2 * Fx], W1[2 * Fx + Fa:]
    V1_x, V1_a = V1[:Fx], V1[Fx:]

    U1bc = jnp.repeat(u @ W1_u, C, axis=0)
    PRe = x @ W1_r + U1bc + (b1 + pre_const)
    PRn = x @ V1_x + (b1n + b2e @ V1_a)
    PCe = x @ W1_c
    A = W2e @ V1_a

    moff = 32 if prev_is_h else 16
    mat = jnp.zeros((MATW,), jnp.float32)
    mat = mat.at[:mhead.size].set(mhead.reshape(-1))
    mat = mat.at[moff:moff + 5 * Hn1].set(A.reshape(-1))

    N = x.shape[0]
    tabr = jnp.concatenate(
        [PRe, PRn, jnp.zeros((N, WRG - 5 - Hn1), jnp.float32)], axis=1)
    tabc = jnp.concatenate(
        [PCe, jnp.zeros((N, WCG - 5), jnp.float32)], axis=1)
    payload, hout = _sc_edge_call(tabr, tabc, row, col, prev, mat, zeros_kf,
                                  Hn1, Wp, prev_is_h, ones_col, emit_h)

    parts = _sc_scatter_call(payload, col2d, zeros)
    S = _combine_halves(parts, N)
    Sh, Sz = S[:, :5], S[:, 5:5 + Hn1]
    if ones_col:
        cnt = S[:, 5 + Hn1:6 + Hn1]
        cnt_inv = 1.0 / jnp.maximum(cnt, 1.0)
        pos = jnp.where(cnt > 0, 1.0, 0.0)
    agg = (Sz * cnt_inv) @ V2 + b2n * pos
    edge_u = (Sh * cnt_inv) @ W2e + b2e * pos

    xu = jnp.repeat(u, C, axis=0)
    x_new = _mlp(jnp.concatenate([x, agg, xu], axis=1), p, prefix + "_n2")
    return x_new, hout, edge_u, cnt_inv, pos


def _global_model(prefix, u, x_new, edge_u, p):
    xbar = x_new.reshape(B, C, -1).mean(axis=1)
    ebar = edge_u.reshape(B, C, -1).mean(axis=1)
    return _mlp(jnp.concatenate([u, xbar, ebar], axis=1), p, prefix + "_g")


def kernel(deepgo, x, u, edge, edge_attr, batch, params):
    p = params
    nB = deepgo.shape[0]
    N = x.shape[0]
    row, col = edge[0], edge[1]
    col2d = col.reshape(-1, G, GU)
    node_dpg = jnp.squeeze(deepgo, 1).reshape(C * nB, 1)
    uu = jnp.squeeze(u[:, :, 1:21], 1)

    zeros24 = jnp.zeros((KB, 24), jnp.float32)
    zeros32 = jnp.zeros((KB, 32), jnp.float32)
    zkf24 = jnp.zeros((KF, 24), jnp.float32)
    zkf32 = jnp.zeros((KF, 32), jnp.float32)
    zero5 = jnp.zeros((5,), jnp.float32)

    W1a_1 = p["ml1_e_w1"][2 * 12]
    ea1d = jnp.squeeze(edge_attr, 1)
    x1, h1, eu1, cnt_inv, pos = _layer(
        "ml1", x, row, col, col2d, ea1d, W1a_1, zero5, uu, p,
        None, None, 12, 1, 15, 24, zeros24, zkf24, False, ones_col=True)
    uu = _global_model("ml1", uu, x1, eu1, p)

    W1a_2 = p["ml2_e_w1"][2 * 20:2 * 20 + 10]
    M2 = p["ml1_e_w2"] @ W1a_2
    c2 = p["ml1_e_b2"] @ W1a_2
    x2, h2, eu2, _, _ = _layer(
        "ml2", x1, row, col, col2d, h1, M2, c2, uu, p,
        cnt_inv, pos, 20, 10, 20, 32, zeros32, zkf32, True)
    uu = _global_model("ml2", uu, x2, eu2, p)

    x2b = jnp.concatenate([x2, node_dpg, x[:, 10:]], axis=1)
    W1a_5 = p["ml5_e_w1"][2 * 23:2 * 23 + 10]
    M5 = p["ml2_e_w2"] @ W1a_5
    c5 = p["ml2_e_b2"] @ W1a_5
    x3, _, _, _, _ = _layer(
        "ml5", x2b, row, col, col2d, h2, M5, c5, uu, p,
        cnt_inv, pos, 23, 10, 23, 32, zeros32, zkf32, True, emit_h=False)
    return x3.reshape(nB, 1, -1)
